# baseline (speedup 1.0000x reference)
import functools

import jax
import jax.numpy as jnp
from jax import lax
from jax.experimental import pallas as pl
from jax.experimental.pallas import tpu as pltpu
from jax.experimental.pallas import tpu_sc as plsc

NLANE = 128
TM = 512
TN = 512
NW = 32
_NC = 2

def _sc_mesh():
    return plsc.VectorSubcoreMesh(
        core_axis_name="c", subcore_axis_name="s",
        num_cores=_NC, num_subcores=NW // _NC)


def _router_body(x_ref, gw_ref, gb_ref, logits_ref, w_ref, e_ref):
    x = x_ref[...]
    logits = (
        jnp.dot(x, gw_ref[...], preferred_element_type=jnp.float32)
        + gb_ref[...]
    )
    logits_ref[...] = logits
    m = jnp.max(logits, axis=1, keepdims=True)
    p = jnp.exp(logits - m)
    probs = p / jnp.sum(p, axis=1, keepdims=True)
    idx = lax.broadcasted_iota(jnp.int32, probs.shape, 1)
    m1 = jnp.max(probs, axis=1, keepdims=True)
    i1 = jnp.min(jnp.where(probs == m1, idx, NLANE), axis=1, keepdims=True)
    p2 = jnp.where(idx == i1, -1.0, probs)
    m2 = jnp.max(p2, axis=1, keepdims=True)
    i2 = jnp.min(jnp.where(p2 == m2, idx, NLANE), axis=1, keepdims=True)
    s = m1 + m2
    w_ref[...] = jnp.where(idx == 0, m1 / s, jnp.where(idx == 1, m2 / s, 0.0))
    e_ref[...] = jnp.where(idx == 0, i1, jnp.where(idx == 1, i2, 0))


def _router(x, gate_w, gate_b):
    t, d = x.shape
    e = gate_w.shape[1]
    gwp = jnp.concatenate(
        [gate_w, jnp.zeros((d, NLANE - e), jnp.float32)], axis=1)
    gbp = jnp.concatenate(
        [gate_b, jnp.full((NLANE - e,), -1e30, jnp.float32)]).reshape(1, NLANE)
    tt = 1024
    logits_p, w_p, e_p = pl.pallas_call(
        _router_body,
        grid=(t // tt,),
        in_specs=[
            pl.BlockSpec((tt, d), lambda i: (i, 0)),
            pl.BlockSpec((d, NLANE), lambda i: (0, 0)),
            pl.BlockSpec((1, NLANE), lambda i: (0, 0)),
        ],
        out_specs=[
            pl.BlockSpec((tt, NLANE), lambda i: (i, 0)),
            pl.BlockSpec((tt, NLANE), lambda i: (i, 0)),
            pl.BlockSpec((tt, NLANE), lambda i: (i, 0)),
        ],
        out_shape=[
            jax.ShapeDtypeStruct((t, NLANE), jnp.float32),
            jax.ShapeDtypeStruct((t, NLANE), jnp.float32),
            jax.ShapeDtypeStruct((t, NLANE), jnp.int32),
        ],
    )(x, gwp, gbp)
    return logits_p[:, :e], w_p, e_p[:, :2]


def _dispatch_body(e_ref, pos_ref, eot_ref, *, num_experts, nt):
    ef = e_ref[...]
    rows = ef.shape[0]
    lane = lax.broadcasted_iota(jnp.int32, (rows, 128), 1)
    u_strict = (lax.broadcasted_iota(jnp.int32, (128, 128), 0)
                < lax.broadcasted_iota(jnp.int32, (128, 128), 1)
                ).astype(jnp.float32)
    l_strict = (lax.broadcasted_iota(jnp.int32, (rows, rows), 1)
                < lax.broadcasted_iota(jnp.int32, (rows, rows), 0)
                ).astype(jnp.float32)
    ones_col = jnp.ones((128, 128), jnp.float32)
    pos = jnp.zeros((rows, 128), jnp.float32)
    start = jnp.float32(0.0)
    tiles_cum = []
    for e in range(num_experts):
        m = (ef == e).astype(jnp.float32)
        inrow = jnp.dot(m, u_strict, preferred_element_type=jnp.float32)
        rowsum = jnp.dot(m, ones_col, preferred_element_type=jnp.float32)
        prevrows = jnp.dot(l_strict, rowsum,
                           preferred_element_type=jnp.float32)
        rank = inrow + prevrows
        pos = pos + m * (start + rank)
        g = jnp.sum(m)
        gpad = jnp.ceil(g / TM) * TM
        start = start + gpad
        tiles_cum.append(start / TM)
    pos_ref[...] = pos.astype(jnp.int32)
    tl = lax.broadcasted_iota(jnp.int32, (8, 128), 1).astype(jnp.float32)
    eot = jnp.zeros((8, 128), jnp.float32)
    for c in tiles_cum:
        eot = eot + (tl >= c).astype(jnp.float32)
    eot = jnp.minimum(eot, num_experts - 1)
    live = (tl < tiles_cum[-1]).astype(jnp.float32)
    first = (tl == 0.0).astype(jnp.float32)
    for c in tiles_cum[:-1]:
        first = jnp.maximum(first, (tl == c).astype(jnp.float32))
    first = first * live
    row = lax.broadcasted_iota(jnp.int32, (8, 128), 0)
    eot_ref[...] = jnp.where(
        row == 0, eot,
        jnp.where(row == 1, live, first)).astype(jnp.int32)


def _dispatch(etop, num_experts, nt):
    t = etop.shape[0]
    rows = (2 * t) // 128
    ef = etop.reshape(rows, 128)
    pos, eot_pad = pl.pallas_call(
        functools.partial(_dispatch_body, num_experts=num_experts, nt=nt),
        out_shape=[
            jax.ShapeDtypeStruct((rows, 128), jnp.int32),
            jax.ShapeDtypeStruct((8, 128), jnp.int32),
        ],
    )(ef)
    pos2 = pos.reshape(t, 2)
    return (pos2[:, 0], pos2[:, 1],
            eot_pad[0, :nt], eot_pad[1, :nt], eot_pad[2, :nt])


def _scatter_sorted(x, pos0, pos1, rows_pad):
    t, d = x.shape
    tpw = t // NW
    ch = 16
    nch = tpw // ch
    p0r = pos0.reshape(NW, nch, ch)
    p1r = pos1.reshape(NW, nch, ch)

    @functools.partial(
        pl.kernel,
        out_type=jax.ShapeDtypeStruct((rows_pad, d), x.dtype),
        mesh=_sc_mesh(),
        scratch_types=[
            pltpu.VMEM((nch, ch), jnp.int32),
            pltpu.VMEM((nch, ch), jnp.int32),
            pltpu.VMEM((ch, d), x.dtype),
            pltpu.SemaphoreType.DMA,
        ],
    )
    def k(x_hbm, p0_hbm, p1_hbm, xs_hbm, p0_v, p1_v, rows_v, sem):
        wid = lax.axis_index("s") * _NC + lax.axis_index("c")
        base = wid * tpw
        pltpu.sync_copy(p0_hbm.at[wid], p0_v)
        pltpu.sync_copy(p1_hbm.at[wid], p1_v)
        for c in range(nch):
            pltpu.sync_copy(x_hbm.at[pl.ds(base + c * ch, ch)], rows_v)
            cp0 = pltpu.async_copy(rows_v, xs_hbm.at[p0_v.at[c]], sem)
            cp1 = pltpu.async_copy(rows_v, xs_hbm.at[p1_v.at[c]], sem)
            cp0.wait()
            cp1.wait()

    return k(x, p0r, p1r)


def _gemm_body(er_ref, lv_ref, fr_ref, xs_ref, w1_ref, w3_ref, w2_ref,
               out_ref, xb_c, w1_c, w3_c, w2_c):
    ti = pl.program_id(0)
    fi = pl.program_id(1)
    nf = pl.num_programs(1)

    @pl.when(lv_ref[ti] == 1)
    def _():
        @pl.when(fi == 0)
        def _():
            xb_c[...] = xs_ref[...].astype(jnp.bfloat16)

        x = xb_c[...]

        @pl.when(fr_ref[ti] == 1)
        def _():
            w1_c[:, pl.ds(fi * TN, TN)] = w1_ref[0].astype(jnp.bfloat16)
            w3_c[:, pl.ds(fi * TN, TN)] = w3_ref[0].astype(jnp.bfloat16)
            w2_c[pl.ds(fi * TN, TN), :] = w2_ref[0].astype(jnp.bfloat16)

        w1b = w1_c[:, pl.ds(fi * TN, TN)]
        w3b = w3_c[:, pl.ds(fi * TN, TN)]
        w2b = w2_c[pl.ds(fi * TN, TN), :]
        h1 = jnp.dot(x, w1b, preferred_element_type=jnp.float32)
        h3 = jnp.dot(x, w3b, preferred_element_type=jnp.float32)
        a = (h1 * lax.logistic(h1) * h3).astype(jnp.bfloat16)
        contrib = jnp.dot(a, w2b, preferred_element_type=jnp.float32)

        @pl.when(fi == 0)
        def _():
            out_ref[...] = contrib

        @pl.when(fi != 0)
        def _():
            out_ref[...] += contrib


def _grouped_mlp(xs, w1, w3, w2, eot, live, first, nt):
    rows_pad, d = xs.shape
    ff = w1.shape[2]
    nf = ff // TN

    def _fidx(ti, fi, fr):
        return jnp.where(fr[ti] == 1, fi, nf - 1)

    grid_spec = pltpu.PrefetchScalarGridSpec(
        num_scalar_prefetch=3,
        grid=(nt, nf),
        in_specs=[
            pl.BlockSpec((TM, d), lambda ti, fi, er, lv, fr: (ti * lv[ti], 0)),
            pl.BlockSpec((1, d, TN),
                         lambda ti, fi, er, lv, fr:
                         (er[ti], 0, _fidx(ti, fi, fr))),
            pl.BlockSpec((1, d, TN),
                         lambda ti, fi, er, lv, fr:
                         (er[ti], 0, _fidx(ti, fi, fr))),
            pl.BlockSpec((1, TN, d),
                         lambda ti, fi, er, lv, fr:
                         (er[ti], _fidx(ti, fi, fr), 0)),
        ],
        out_specs=pl.BlockSpec((TM, d), lambda ti, fi, er, lv, fr: (ti, 0)),
        scratch_shapes=[
            pltpu.VMEM((TM, d), jnp.bfloat16),
            pltpu.VMEM((d, ff), jnp.bfloat16),
            pltpu.VMEM((d, ff), jnp.bfloat16),
            pltpu.VMEM((ff, d), jnp.bfloat16),
        ],
    )
    return pl.pallas_call(
        _gemm_body,
        grid_spec=grid_spec,
        out_shape=jax.ShapeDtypeStruct((rows_pad, d), jnp.float32),
        compiler_params=pltpu.CompilerParams(
            dimension_semantics=("arbitrary", "arbitrary")),
    )(eot, live, first, xs, w1, w3, w2)


def _gather_pair(ys, pos0, pos1):
    t = pos0.shape[0]
    d = ys.shape[1]
    tpw = t // NW
    ch = 64
    nch = tpw // ch

    @functools.partial(
        pl.kernel,
        out_type=(
            jax.ShapeDtypeStruct((t, d), jnp.float32),
            jax.ShapeDtypeStruct((t, d), jnp.float32),
        ),
        mesh=_sc_mesh(),
        scratch_types=[
            pltpu.VMEM((ch,), jnp.int32),
            pltpu.VMEM((ch, d), jnp.float32),
            pltpu.SemaphoreType.DMA,
        ],
    )
    def k(ys_hbm, p0_hbm, p1_hbm, y0_hbm, y1_hbm, idx_v, buf_v, sem):
        wid = lax.axis_index("s") * _NC + lax.axis_index("c")
        base = wid * tpw
        for c in range(nch):
            lo = base + c * ch
            pltpu.sync_copy(p0_hbm.at[pl.ds(lo, ch)], idx_v)
            pltpu.async_copy(ys_hbm.at[idx_v], buf_v, sem).wait()
            pltpu.sync_copy(buf_v, y0_hbm.at[pl.ds(lo, ch)])
            pltpu.sync_copy(p1_hbm.at[pl.ds(lo, ch)], idx_v)
            pltpu.async_copy(ys_hbm.at[idx_v], buf_v, sem).wait()
            pltpu.sync_copy(buf_v, y1_hbm.at[pl.ds(lo, ch)])

    return k(ys, pos0, pos1)


def _combine_body(y0_ref, y1_ref, w_ref, o_ref):
    o_ref[...] = (y0_ref[...] * w_ref[:, 0:1] + y1_ref[...] * w_ref[:, 1:2])


def _combine(y0, y1, w_p):
    t, d = y0.shape
    tt = 512
    return pl.pallas_call(
        _combine_body,
        grid=(t // tt,),
        in_specs=[
            pl.BlockSpec((tt, d), lambda i: (i, 0)),
            pl.BlockSpec((tt, d), lambda i: (i, 0)),
            pl.BlockSpec((tt, NLANE), lambda i: (i, 0)),
        ],
        out_specs=pl.BlockSpec((tt, d), lambda i: (i, 0)),
        out_shape=jax.ShapeDtypeStruct((t, d), jnp.float32),
    )(y0, y1, w_p)


def kernel(hidden_states, gate_w, gate_b, w1, w3, w2):
    b, s, d = hidden_states.shape
    t = b * s
    num_experts = gate_b.shape[0]
    nt = (2 * t) // TM + num_experts
    rows_pad = nt * TM

    x = hidden_states.reshape(t, d)
    logits, w_p, etop = _router(x, gate_w, gate_b)
    pos0, pos1, eot, live, first = _dispatch(etop, num_experts, nt)
    xs = _scatter_sorted(x, pos0, pos1, rows_pad)
    ys = _grouped_mlp(xs, w1, w3, w2, eot, live, first, nt)
    y0, y1 = _gather_pair(ys, pos0, pos1)
    final = _combine(y0, y1, w_p)
    return final.reshape(b, s, d), logits


# speedup vs baseline: 1.1251x; 1.1251x over previous
import functools

import jax
import jax.numpy as jnp
from jax import lax
from jax.experimental import pallas as pl
from jax.experimental.pallas import tpu as pltpu
from jax.experimental.pallas import tpu_sc as plsc

NLANE = 128
TM = 512
TN = 1024
NW = 32
_NC = 2

def _sc_mesh():
    return plsc.VectorSubcoreMesh(
        core_axis_name="c", subcore_axis_name="s",
        num_cores=_NC, num_subcores=NW // _NC)


def _router_body(x_ref, gw_ref, gb_ref, logits_ref, w_ref, e_ref):
    x = x_ref[...]
    logits = (
        jnp.dot(x, gw_ref[...], preferred_element_type=jnp.float32)
        + gb_ref[...]
    )
    logits_ref[...] = logits
    m = jnp.max(logits, axis=1, keepdims=True)
    p = jnp.exp(logits - m)
    probs = p / jnp.sum(p, axis=1, keepdims=True)
    idx = lax.broadcasted_iota(jnp.int32, probs.shape, 1)
    m1 = jnp.max(probs, axis=1, keepdims=True)
    i1 = jnp.min(jnp.where(probs == m1, idx, NLANE), axis=1, keepdims=True)
    p2 = jnp.where(idx == i1, -1.0, probs)
    m2 = jnp.max(p2, axis=1, keepdims=True)
    i2 = jnp.min(jnp.where(p2 == m2, idx, NLANE), axis=1, keepdims=True)
    s = m1 + m2
    w_ref[...] = jnp.where(idx == 0, m1 / s, jnp.where(idx == 1, m2 / s, 0.0))
    e_ref[...] = jnp.where(idx == 0, i1, jnp.where(idx == 1, i2, 0))


def _router(x, gate_w, gate_b):
    t, d = x.shape
    e = gate_w.shape[1]
    gwp = jnp.concatenate(
        [gate_w, jnp.zeros((d, NLANE - e), jnp.float32)], axis=1)
    gbp = jnp.concatenate(
        [gate_b, jnp.full((NLANE - e,), -1e30, jnp.float32)]).reshape(1, NLANE)
    tt = 1024
    logits_p, w_p, e_p = pl.pallas_call(
        _router_body,
        grid=(t // tt,),
        in_specs=[
            pl.BlockSpec((tt, d), lambda i: (i, 0)),
            pl.BlockSpec((d, NLANE), lambda i: (0, 0)),
            pl.BlockSpec((1, NLANE), lambda i: (0, 0)),
        ],
        out_specs=[
            pl.BlockSpec((tt, NLANE), lambda i: (i, 0)),
            pl.BlockSpec((tt, NLANE), lambda i: (i, 0)),
            pl.BlockSpec((tt, NLANE), lambda i: (i, 0)),
        ],
        out_shape=[
            jax.ShapeDtypeStruct((t, NLANE), jnp.float32),
            jax.ShapeDtypeStruct((t, NLANE), jnp.float32),
            jax.ShapeDtypeStruct((t, NLANE), jnp.int32),
        ],
    )(x, gwp, gbp)
    return logits_p[:, :e], w_p, e_p[:, :2]


def _dispatch_body(e_ref, pos_ref, eot_ref, *, num_experts, nt):
    ef = e_ref[...]
    rows = ef.shape[0]
    lane = lax.broadcasted_iota(jnp.int32, (rows, 128), 1)
    u_strict = (lax.broadcasted_iota(jnp.int32, (128, 128), 0)
                < lax.broadcasted_iota(jnp.int32, (128, 128), 1)
                ).astype(jnp.float32)
    l_strict = (lax.broadcasted_iota(jnp.int32, (rows, rows), 1)
                < lax.broadcasted_iota(jnp.int32, (rows, rows), 0)
                ).astype(jnp.float32)
    ones_col = jnp.ones((128, 128), jnp.float32)
    pos = jnp.zeros((rows, 128), jnp.float32)
    start = jnp.float32(0.0)
    tiles_cum = []
    for e in range(num_experts):
        m = (ef == e).astype(jnp.float32)
        inrow = jnp.dot(m, u_strict, preferred_element_type=jnp.float32)
        rowsum = jnp.dot(m, ones_col, preferred_element_type=jnp.float32)
        prevrows = jnp.dot(l_strict, rowsum,
                           preferred_element_type=jnp.float32)
        rank = inrow + prevrows
        pos = pos + m * (start + rank)
        g = jnp.sum(m)
        gpad = jnp.ceil(g / TM) * TM
        start = start + gpad
        tiles_cum.append(start / TM)
    pos_ref[...] = pos.astype(jnp.int32)
    tl = lax.broadcasted_iota(jnp.int32, (8, 128), 1).astype(jnp.float32)
    eot = jnp.zeros((8, 128), jnp.float32)
    for c in tiles_cum:
        eot = eot + (tl >= c).astype(jnp.float32)
    eot = jnp.minimum(eot, num_experts - 1)
    live = (tl < tiles_cum[-1]).astype(jnp.float32)
    first = (tl == 0.0).astype(jnp.float32)
    for c in tiles_cum[:-1]:
        first = jnp.maximum(first, (tl == c).astype(jnp.float32))
    first = first * live
    row = lax.broadcasted_iota(jnp.int32, (8, 128), 0)
    eot_ref[...] = jnp.where(
        row == 0, eot,
        jnp.where(row == 1, live, first)).astype(jnp.int32)


def _dispatch(etop, num_experts, nt):
    t = etop.shape[0]
    rows = (2 * t) // 128
    ef = etop.reshape(rows, 128)
    pos, eot_pad = pl.pallas_call(
        functools.partial(_dispatch_body, num_experts=num_experts, nt=nt),
        out_shape=[
            jax.ShapeDtypeStruct((rows, 128), jnp.int32),
            jax.ShapeDtypeStruct((8, 128), jnp.int32),
        ],
    )(ef)
    pos2 = pos.reshape(t, 2)
    return (pos2[:, 0], pos2[:, 1],
            eot_pad[0, :nt], eot_pad[1, :nt], eot_pad[2, :nt])


def _scatter_sorted(x, pos0, pos1, rows_pad):
    t, d = x.shape
    tpw = t // NW
    ch = 16
    nch = tpw // ch
    p0r = pos0.reshape(NW, nch, ch)
    p1r = pos1.reshape(NW, nch, ch)

    @functools.partial(
        pl.kernel,
        out_type=jax.ShapeDtypeStruct((rows_pad, d), x.dtype),
        mesh=_sc_mesh(),
        scratch_types=[
            pltpu.VMEM((nch, ch), jnp.int32),
            pltpu.VMEM((nch, ch), jnp.int32),
            pltpu.VMEM((ch, d), x.dtype),
            pltpu.SemaphoreType.DMA,
        ],
    )
    def k(x_hbm, p0_hbm, p1_hbm, xs_hbm, p0_v, p1_v, rows_v, sem):
        wid = lax.axis_index("s") * _NC + lax.axis_index("c")
        base = wid * tpw
        pltpu.sync_copy(p0_hbm.at[wid], p0_v)
        pltpu.sync_copy(p1_hbm.at[wid], p1_v)
        for c in range(nch):
            pltpu.sync_copy(x_hbm.at[pl.ds(base + c * ch, ch)], rows_v)
            cp0 = pltpu.async_copy(rows_v, xs_hbm.at[p0_v.at[c]], sem)
            cp1 = pltpu.async_copy(rows_v, xs_hbm.at[p1_v.at[c]], sem)
            cp0.wait()
            cp1.wait()

    return k(x, p0r, p1r)


def _gemm_body(er_ref, lv_ref, fr_ref, xs_ref, w1_ref, w3_ref, w2_ref,
               out_ref, xb_c, w1_c, w3_c):
    ti = pl.program_id(0)
    fi = pl.program_id(1)

    @pl.when(lv_ref[ti] == 1)
    def _():
        @pl.when(fi == 0)
        def _():
            xb_c[...] = xs_ref[...].astype(jnp.bfloat16)

        x = xb_c[...]

        @pl.when(fr_ref[ti] == 1)
        def _():
            w1_c[:, pl.ds(fi * TN, TN)] = w1_ref[0].astype(jnp.bfloat16)
            w3_c[:, pl.ds(fi * TN, TN)] = w3_ref[0].astype(jnp.bfloat16)

        w1b = w1_c[:, pl.ds(fi * TN, TN)]
        w3b = w3_c[:, pl.ds(fi * TN, TN)]
        h1 = jnp.dot(x, w1b, preferred_element_type=jnp.float32)
        h3 = jnp.dot(x, w3b, preferred_element_type=jnp.float32)
        a = h1 * lax.logistic(h1) * h3
        contrib = jnp.dot(a, w2_ref[0], preferred_element_type=jnp.float32)

        @pl.when(fi == 0)
        def _():
            out_ref[...] = contrib

        @pl.when(fi != 0)
        def _():
            out_ref[...] += contrib


def _grouped_mlp(xs, w1, w3, w2, eot, live, first, nt):
    rows_pad, d = xs.shape
    ff = w1.shape[2]
    nf = ff // TN

    def _fidx(ti, fi, fr):
        return jnp.where(fr[ti] == 1, fi, nf - 1)

    grid_spec = pltpu.PrefetchScalarGridSpec(
        num_scalar_prefetch=3,
        grid=(nt, nf),
        in_specs=[
            pl.BlockSpec((TM, d), lambda ti, fi, er, lv, fr: (ti * lv[ti], 0)),
            pl.BlockSpec((1, d, TN),
                         lambda ti, fi, er, lv, fr:
                         (er[ti], 0, _fidx(ti, fi, fr))),
            pl.BlockSpec((1, d, TN),
                         lambda ti, fi, er, lv, fr:
                         (er[ti], 0, _fidx(ti, fi, fr))),
            pl.BlockSpec((1, TN, d),
                         lambda ti, fi, er, lv, fr:
                         (er[ti], fi * lv[ti], 0)),
        ],
        out_specs=pl.BlockSpec((TM, d), lambda ti, fi, er, lv, fr: (ti, 0)),
        scratch_shapes=[
            pltpu.VMEM((TM, d), jnp.bfloat16),
            pltpu.VMEM((d, ff), jnp.bfloat16),
            pltpu.VMEM((d, ff), jnp.bfloat16),
        ],
    )
    return pl.pallas_call(
        _gemm_body,
        grid_spec=grid_spec,
        out_shape=jax.ShapeDtypeStruct((rows_pad, d), jnp.float32),
        compiler_params=pltpu.CompilerParams(
            dimension_semantics=("arbitrary", "arbitrary")),
    )(eot, live, first, xs, w1, w3, w2)


def _gather_pair(ys, pos0, pos1):
    t = pos0.shape[0]
    d = ys.shape[1]
    tpw = t // NW
    ch = 64
    nch = tpw // ch

    @functools.partial(
        pl.kernel,
        out_type=(
            jax.ShapeDtypeStruct((t, d), jnp.float32),
            jax.ShapeDtypeStruct((t, d), jnp.float32),
        ),
        mesh=_sc_mesh(),
        scratch_types=[
            pltpu.VMEM((ch,), jnp.int32),
            pltpu.VMEM((ch, d), jnp.float32),
            pltpu.SemaphoreType.DMA,
        ],
    )
    def k(ys_hbm, p0_hbm, p1_hbm, y0_hbm, y1_hbm, idx_v, buf_v, sem):
        wid = lax.axis_index("s") * _NC + lax.axis_index("c")
        base = wid * tpw
        for c in range(nch):
            lo = base + c * ch
            pltpu.sync_copy(p0_hbm.at[pl.ds(lo, ch)], idx_v)
            pltpu.async_copy(ys_hbm.at[idx_v], buf_v, sem).wait()
            pltpu.sync_copy(buf_v, y0_hbm.at[pl.ds(lo, ch)])
            pltpu.sync_copy(p1_hbm.at[pl.ds(lo, ch)], idx_v)
            pltpu.async_copy(ys_hbm.at[idx_v], buf_v, sem).wait()
            pltpu.sync_copy(buf_v, y1_hbm.at[pl.ds(lo, ch)])

    return k(ys, pos0, pos1)


def _combine_body(y0_ref, y1_ref, w_ref, o_ref):
    o_ref[...] = (y0_ref[...] * w_ref[:, 0:1] + y1_ref[...] * w_ref[:, 1:2])


def _combine(y0, y1, w_p):
    t, d = y0.shape
    tt = 512
    return pl.pallas_call(
        _combine_body,
        grid=(t // tt,),
        in_specs=[
            pl.BlockSpec((tt, d), lambda i: (i, 0)),
            pl.BlockSpec((tt, d), lambda i: (i, 0)),
            pl.BlockSpec((tt, NLANE), lambda i: (i, 0)),
        ],
        out_specs=pl.BlockSpec((tt, d), lambda i: (i, 0)),
        out_shape=jax.ShapeDtypeStruct((t, d), jnp.float32),
    )(y0, y1, w_p)


def kernel(hidden_states, gate_w, gate_b, w1, w3, w2):
    b, s, d = hidden_states.shape
    t = b * s
    num_experts = gate_b.shape[0]
    nt = (2 * t) // TM + num_experts
    rows_pad = nt * TM

    x = hidden_states.reshape(t, d)
    logits, w_p, etop = _router(x, gate_w, gate_b)
    pos0, pos1, eot, live, first = _dispatch(etop, num_experts, nt)
    xs = _scatter_sorted(x, pos0, pos1, rows_pad)
    ys = _grouped_mlp(xs, w1, w3, w2, eot, live, first, nt)
    y0, y1 = _gather_pair(ys, pos0, pos1)
    final = _combine(y0, y1, w_p)
    return final.reshape(b, s, d), logits
